# baseline (speedup 1.0000x reference)
import functools

import jax
import jax.numpy as jnp
from jax import lax
from jax.experimental import pallas as pl
from jax.experimental.pallas import tpu as pltpu
from jax.experimental.pallas import tpu_sc as plsc

_V = 1000000
_B = 4096
_L = 200
_D = 64
_C = 20
_NC = 2
_NS = 16
_NW = _NC * _NS
_SPW = _B // _NW
_H0 = 96
_H1 = _L - _H0


def _make_gather_pool():
    mesh = plsc.VectorSubcoreMesh(
        core_axis_name="c", subcore_axis_name="s",
        num_cores=_NC, num_subcores=_NS,
    )

    @functools.partial(
        pl.kernel,
        out_type=jax.ShapeDtypeStruct((_B, _D), jnp.float32),
        mesh=mesh,
        name="gather_pool",
        compiler_params=pltpu.CompilerParams(use_tc_tiling_on_sc=False),
        scratch_types=[
            pltpu.VMEM((_SPW, _L), jnp.int32),
            pltpu.VMEM((4, _L, _D), jnp.float32),
            pltpu.VMEM((_SPW, _D), jnp.float32),
            pltpu.SemaphoreType.DMA,
            pltpu.SemaphoreType.DMA,
            pltpu.SemaphoreType.DMA,
            pltpu.SemaphoreType.DMA,
        ],
    )
    def gather_pool(x_hbm, table_hbm, out_hbm,
                    idx_v, rows_v, pooled_v, s0, s1, s2, s3):
        wid = lax.axis_index("s") * _NC + lax.axis_index("c")
        sems = (s0, s1, s2, s3)
        pltpu.sync_copy(x_hbm.at[pl.ds(wid * _SPW, _SPW)], idx_v)

        def fire(i, p):
            @pl.when(i < _SPW)
            def _():
                pltpu.async_copy(table_hbm.at[idx_v.at[i, pl.ds(0, _H0)]],
                                 rows_v.at[p, pl.ds(0, _H0)], sems[p])
                pltpu.async_copy(table_hbm.at[idx_v.at[i, pl.ds(_H0, _H1)]],
                                 rows_v.at[p, pl.ds(_H0, _H1)], sems[p])

        def wait(i, p):
            pltpu.make_async_copy(table_hbm.at[idx_v.at[i, pl.ds(0, _H0)]],
                                  rows_v.at[p, pl.ds(0, _H0)], sems[p]).wait()
            pltpu.make_async_copy(table_hbm.at[idx_v.at[i, pl.ds(_H0, _H1)]],
                                  rows_v.at[p, pl.ds(_H0, _H1)], sems[p]).wait()

        def consume(i, p):
            def body(l, accs):
                return tuple(a + rows_v[p, l, pl.ds(q * 16, 16)]
                             for q, a in enumerate(accs))
            accs = tuple(jnp.zeros((16,), jnp.float32) for _ in range(4))
            accs = lax.fori_loop(0, _L, body, accs, unroll=8)
            for q in range(4):
                pooled_v[i, pl.ds(q * 16, 16)] = accs[q]

        for p in range(4):
            fire(p, p)

        def step(k, carry):
            for p in range(4):
                i = 4 * k + p
                wait(i, p)
                consume(i, p)
                fire(i + 4, p)
            return carry

        lax.fori_loop(0, _SPW // 4, step, 0)
        pltpu.sync_copy(pooled_v, out_hbm.at[pl.ds(wid * _SPW, _SPW)])

    return gather_pool


_gather_pool = _make_gather_pool()


def _classifier_kernel(p_ref, w_ref, b_ref, o_ref):
    o_ref[...] = (
        jnp.dot(p_ref[...], w_ref[...], preferred_element_type=jnp.float32)
        + b_ref[...]
    )


def _classifier(pooled, W, b2d):
    return pl.pallas_call(
        _classifier_kernel,
        out_shape=jax.ShapeDtypeStruct((_B, _C), jnp.float32),
    )(pooled, W, b2d)


@jax.jit
def kernel(x, table, W, b):
    table2 = jnp.reshape(table, (_V // 2, 2 * _D))
    table2 = jax.lax.optimization_barrier(table2)
    table3 = jnp.reshape(table2, (_V, _D))
    pooled = _gather_pool(x.astype(jnp.int32), table3)
    return _classifier(pooled, W, b.reshape(1, _C))
